# baseline (speedup 1.0000x reference)
import functools

import jax
import jax.numpy as jnp
from jax import lax
from jax.experimental import pallas as pl
from jax.experimental.pallas import tpu as pltpu
from jax.experimental.pallas import tpu_sc as plsc

N = 10000
K = 16
E = N * K
IN_CHS = 128
C0 = 64


def _lrelu(v):
    return jnp.where(v >= 0, v, 0.2 * v)


def _h0_body(x_ref, pos_ref, w_ref, b_ref, o_ref):
    h = jnp.dot(x_ref[...], w_ref[...], preferred_element_type=jnp.float32)
    o_ref[:, :C0] = _lrelu(h + b_ref[...])
    o_ref[:, C0:C0 + 8] = pos_ref[...]
    o_ref[:, C0 + 8:] = jnp.zeros((x_ref.shape[0], 128 - C0 - 8), jnp.float32)


def _h0cat(x, pos8, W_in, b_in):
    B = 2000
    return pl.pallas_call(
        _h0_body,
        grid=(N // B,),
        in_specs=[
            pl.BlockSpec((B, IN_CHS), lambda i: (i, 0)),
            pl.BlockSpec((B, 8), lambda i: (i, 0)),
            pl.BlockSpec((IN_CHS, C0), lambda i: (0, 0)),
            pl.BlockSpec((1, C0), lambda i: (0, 0)),
        ],
        out_specs=pl.BlockSpec((B, 128), lambda i: (i, 0)),
        out_shape=jax.ShapeDtypeStruct((N, 128), jnp.float32),
    )(x, pos8, W_in, b_in.reshape(1, C0))


_KNN_Q = 400
_NG = 640


def _knn_body(posq_ref, post3_ref, o_ref):
    f32 = jnp.float32
    pq = posq_ref[...]
    sqq = jnp.sum(pq * pq, axis=1, keepdims=True)
    planes = []
    for r in range(K):
        pt = post3_ref[r]
        sqa = jnp.sum(pt * pt, axis=0, keepdims=True)
        mm = jnp.dot(pq, pt, preferred_element_type=f32)
        planes.append((sqq - 2.0 * mm) + sqa)

    def sweep(excl):
        v = jnp.full((_KNN_Q, _NG), jnp.inf, f32)
        rr = jnp.zeros((_KNN_Q, _NG), jnp.int32)
        for r in range(K):
            pr = planes[r]
            for ex in excl:
                pr = jnp.where(ex == r, jnp.inf, pr)
            take = pr < v
            v = jnp.where(take, pr, v)
            rr = jnp.where(take, r, rr)
        return v, rr

    v1, r1 = sweep([])
    v2, r2 = sweep([r1])
    v3, r3 = sweep([r1, r2])
    v4, r4 = sweep([r1, r2, r3])

    iota = lax.broadcasted_iota(jnp.int32, (_KNN_Q, _NG), 1)
    inf = jnp.full((_KNN_Q, _NG), jnp.inf, f32)
    fv, fe = v1, 16 * iota + r1
    q2v, q2e = v2, 16 * iota + r2
    q3v, q3e = v3, 16 * iota + r3
    q4v, q4e = v4, 16 * iota + r4
    for t in range(K):
        am = jnp.argmin(fv, axis=1).astype(jnp.int32)[:, None]
        msk = iota == am
        elem = jnp.min(jnp.where(msk, fe, jnp.int32(2 ** 30)),
                       axis=1, keepdims=True)
        o_ref[:, t:t + 1] = elem
        fv = jnp.where(msk, q2v, fv)
        fe = jnp.where(msk, q2e, fe)
        q2v = jnp.where(msk, q3v, q2v)
        q2e = jnp.where(msk, q3e, q2e)
        q3v = jnp.where(msk, q4v, q3v)
        q3e = jnp.where(msk, q4e, q3e)
        q4v = jnp.where(msk, inf, q4v)

    @pl.when(jnp.any(fv == jnp.inf))
    def _fallback():
        curs = list(planes)
        for t in range(K):
            v = curs[0]
            rr = jnp.zeros((_KNN_Q, _NG), jnp.int32)
            for r in range(1, K):
                take = curs[r] < v
                v = jnp.where(take, curs[r], v)
                rr = jnp.where(take, r, rr)
            am = jnp.argmin(v, axis=1).astype(jnp.int32)[:, None]
            msk = iota == am
            rsel = jnp.min(jnp.where(msk, rr, K), axis=1, keepdims=True)
            o_ref[:, t:t + 1] = 16 * am + rsel
            if t < K - 1:
                for r in range(K):
                    curs[r] = jnp.where(msk & (rsel == r), jnp.inf, curs[r])


def _knn(pos8, posT3):
    return pl.pallas_call(
        _knn_body,
        grid=(N // _KNN_Q,),
        in_specs=[
            pl.BlockSpec((_KNN_Q, 8), lambda i: (i, 0)),
            pl.BlockSpec((K, 8, _NG), lambda i: (0, 0, 0)),
        ],
        out_specs=pl.BlockSpec((_KNN_Q, K), lambda i: (i, 0)),
        out_shape=jax.ShapeDtypeStruct((N, K), jnp.int32),
    )(pos8, posT3)


_GCH = 128
_PER_W = E // 32
_BURST = 4


def _make_gather():
    mesh = plsc.VectorSubcoreMesh(core_axis_name="c", subcore_axis_name="s")

    @functools.partial(
        pl.kernel,
        mesh=mesh,
        out_type=jax.ShapeDtypeStruct((E, 128), jnp.float32),
        scratch_types=[
            pltpu.VMEM((_PER_W,), jnp.int32),
            pltpu.VMEM((_BURST * _GCH, 128), jnp.float32),
            pltpu.SemaphoreType.DMA,
        ],
    )
    def gk(tab_hbm, idx_hbm, out_hbm, idx_v, rows_v, sem):
        wid = lax.axis_index("s") * 2 + lax.axis_index("c")
        base = pl.multiple_of(wid * _PER_W, 8)
        pltpu.sync_copy(idx_hbm.at[pl.ds(base, _PER_W)], idx_v)

        def body(i, carry):
            w = jnp.minimum(i * (_BURST * _GCH), _PER_W - _BURST * _GCH)
            w = pl.multiple_of(w, 8)
            cps = []
            for b in range(_BURST):
                off = pl.multiple_of(w + b * _GCH, 8)
                cps.append(pltpu.async_copy(
                    tab_hbm.at[idx_v.at[pl.ds(off, _GCH)]],
                    rows_v.at[pl.ds(b * _GCH, _GCH)], sem))
            for cp in cps:
                cp.wait()
            pltpu.sync_copy(rows_v, out_hbm.at[pl.ds(base + w, _BURST * _GCH)])
            return carry

        lax.fori_loop(0, (_PER_W + _BURST * _GCH - 1) // (_BURST * _GCH),
                      body, 0)

    return gk


def _layer_body(c, emit_posj, *refs):
    f32 = jnp.float32
    if emit_posj:
        (pos_ref, g_ref, wppi_ref, wppj_ref, wpd_ref, bp_ref, wa11_ref,
         wa12_ref, wa21_ref, wa22_ref, ba1_ref, ba2_ref, wo1_ref, wo2_ref,
         bo_ref, o_ref, posj_out_ref) = refs
        posj_in_ref = None
    else:
        (pos_ref, g_ref, posj_in_ref, wppi_ref, wppj_ref, wpd_ref, bp_ref,
         wa11_ref, wa12_ref, wa21_ref, wa22_ref, ba1_ref, ba2_ref, wo1_ref,
         wo2_ref, bo_ref, o_ref) = refs
        posj_out_ref = None
    pi = pos_ref[...]
    wppi = wppi_ref[...]
    wppj = wppj_ref[...]
    wpd = wpd_ref[...]
    bp = bp_ref[...]
    wa11 = wa11_ref[...]
    wa12 = wa12_ref[...]
    wa21 = wa21_ref[...]
    wa22 = wa22_ref[...]
    ba1 = ba1_ref[...]
    ba2 = ba2_ref[...]

    pes, aas, abs_ = [], [], []
    for j in range(K):
        hj = g_ref[j, :, 0:c]
        if emit_posj:
            pj = g_ref[j, :, c:c + 8]
            posj_out_ref[j] = pj
        else:
            pj = posj_in_ref[j]
        diff = pi - pj
        dist = jnp.sqrt(jnp.sum(diff * diff, axis=1, keepdims=True) + 1e-12)
        pe = _lrelu(jnp.dot(pi, wppi, preferred_element_type=f32)
                    + jnp.dot(pj, wppj, preferred_element_type=f32)
                    + dist * wpd + bp)
        aa = (jnp.dot(hj, wa11, preferred_element_type=f32)
              + jnp.dot(pe, wa21, preferred_element_type=f32) + ba1)
        ab = (jnp.dot(hj, wa12, preferred_element_type=f32)
              + jnp.dot(pe, wa22, preferred_element_type=f32) + ba2)
        pes.append(pe)
        aas.append(aa)
        abs_.append(ab)

    ma = aas[0]
    mb = abs_[0]
    for j in range(1, K):
        ma = jnp.maximum(ma, aas[j])
        mb = jnp.maximum(mb, abs_[j])

    sa = jnp.zeros_like(ma)
    sb = jnp.zeros_like(mb)
    a1 = jnp.zeros_like(ma)
    a2 = jnp.zeros_like(mb)
    for j in range(K):
        ea = jnp.exp(aas[j] - ma)
        eb = jnp.exp(abs_[j] - mb)
        sa = sa + ea
        sb = sb + eb
        a1 = a1 + ea * g_ref[j, :, 0:c]
        a2 = a2 + eb * pes[j]

    agg1 = a1 / (sa + 1e-12)
    agg2 = a2 / (sb + 1e-12)
    out = _lrelu(jnp.dot(agg1, wo1_ref[...], preferred_element_type=f32)
                 + jnp.dot(agg2, wo2_ref[...], preferred_element_type=f32)
                 + bo_ref[...])
    o_ref[...] = out


def _layer(c, emit_posj, pos8, g, posj, Wp, bp, Wa, ba, Wo, bo):
    ch = 2 * c
    B = 400
    wppi = jnp.zeros((8, c), jnp.float32).at[0:3].set(Wp[0:3] + Wp[6:9])
    wppj = jnp.zeros((8, c), jnp.float32).at[0:3].set(Wp[3:6] - Wp[6:9])
    wpd = Wp[9:10]
    body = functools.partial(_layer_body, c, emit_posj)
    wspecs = [
        pl.BlockSpec((8, c), lambda i: (0, 0)),
        pl.BlockSpec((8, c), lambda i: (0, 0)),
        pl.BlockSpec((1, c), lambda i: (0, 0)),
        pl.BlockSpec((1, c), lambda i: (0, 0)),
        pl.BlockSpec((c, c), lambda i: (0, 0)),
        pl.BlockSpec((c, c), lambda i: (0, 0)),
        pl.BlockSpec((c, c), lambda i: (0, 0)),
        pl.BlockSpec((c, c), lambda i: (0, 0)),
        pl.BlockSpec((1, c), lambda i: (0, 0)),
        pl.BlockSpec((1, c), lambda i: (0, 0)),
        pl.BlockSpec((c, ch), lambda i: (0, 0)),
        pl.BlockSpec((c, ch), lambda i: (0, 0)),
        pl.BlockSpec((1, ch), lambda i: (0, 0)),
    ]
    wvals = (wppi, wppj, wpd, bp.reshape(1, c),
             Wa[:c, :c], Wa[:c, c:], Wa[c:, :c], Wa[c:, c:],
             ba[:c].reshape(1, c), ba[c:].reshape(1, c),
             Wo[:c], Wo[c:], bo.reshape(1, ch))
    in_specs = [
        pl.BlockSpec((B, 8), lambda i: (i, 0)),
        pl.BlockSpec((K, B, 128), lambda i: (0, i, 0)),
    ]
    ins = [pos8, g]
    if not emit_posj:
        in_specs.append(pl.BlockSpec((K, B, 8), lambda i: (0, i, 0)))
        ins.append(posj)
    in_specs += wspecs
    ins += wvals
    if emit_posj:
        out_specs = [
            pl.BlockSpec((B, ch), lambda i: (i, 0)),
            pl.BlockSpec((K, B, 8), lambda i: (0, i, 0)),
        ]
        out_shape = [
            jax.ShapeDtypeStruct((N, ch), jnp.float32),
            jax.ShapeDtypeStruct((K, N, 8), jnp.float32),
        ]
    else:
        out_specs = pl.BlockSpec((B, ch), lambda i: (i, 0))
        out_shape = jax.ShapeDtypeStruct((N, ch), jnp.float32)
    return pl.pallas_call(
        body,
        grid=(N // B,),
        in_specs=in_specs,
        out_specs=out_specs,
        out_shape=out_shape,
    )(*ins)


def _final_body(h_ref, x_ref, wmo_ref, wsc_ref, b_ref, o_ref):
    f32 = jnp.float32
    v = (jnp.dot(h_ref[...], wmo_ref[...], preferred_element_type=f32)
         + jnp.dot(x_ref[...], wsc_ref[...], preferred_element_type=f32)
         + b_ref[...])
    o_ref[...] = _lrelu(v)


def _final(h2, x, W_mo, W_sc, bsum):
    B = 2000
    EMB = 512
    return pl.pallas_call(
        _final_body,
        grid=(N // B,),
        in_specs=[
            pl.BlockSpec((B, 256), lambda i: (i, 0)),
            pl.BlockSpec((B, IN_CHS), lambda i: (i, 0)),
            pl.BlockSpec((256, EMB), lambda i: (0, 0)),
            pl.BlockSpec((IN_CHS, EMB), lambda i: (0, 0)),
            pl.BlockSpec((1, EMB), lambda i: (0, 0)),
        ],
        out_specs=pl.BlockSpec((B, EMB), lambda i: (i, 0)),
        out_shape=jax.ShapeDtypeStruct((N, EMB), jnp.float32),
    )(h2, x, W_mo, W_sc, bsum.reshape(1, EMB))


def _gather(tab, idxf):
    return _make_gather()(tab, idxf).reshape(K, N, 128)


def kernel(x, pos, batch, W_in, b_in, W_sc, b_sc, Wp0, bp0, Wa0, ba0, Wo0,
           bo0, Wp1, bp1, Wa1, ba1, Wo1, bo1, W_mo, b_mo):
    pos8 = jnp.pad(pos, ((0, 0), (0, 5)))
    posP = jnp.pad(pos8, ((0, 16 * _NG - N), (0, 0)), constant_values=1e3)
    posT3 = posP.reshape(_NG, K, 8).transpose(1, 2, 0)
    idx = _knn(pos8, posT3)
    idxf = idx.T.reshape(-1)
    h0cat = _h0cat(x, pos8, W_in, b_in)
    g0 = _gather(h0cat, idxf)
    h1, posj = _layer(C0, True, pos8, g0, None, Wp0, bp0, Wa0, ba0, Wo0, bo0)
    g1 = _gather(h1, idxf)
    h2 = _layer(2 * C0, False, pos8, g1, posj, Wp1, bp1, Wa1, ba1, Wo1, bo1)
    return _final(h2, x, W_mo, W_sc, b_mo + b_sc)


# speedup vs baseline: 1.1894x; 1.1894x over previous
import functools

import jax
import jax.numpy as jnp
from jax import lax
from jax.experimental import pallas as pl
from jax.experimental.pallas import tpu as pltpu
from jax.experimental.pallas import tpu_sc as plsc

N = 10000
K = 16
E = N * K
IN_CHS = 128
C0 = 64


def _lrelu(v):
    return jnp.where(v >= 0, v, 0.2 * v)


def _h0_body(x_ref, pos_ref, w_ref, b_ref, o_ref):
    h = jnp.dot(x_ref[...], w_ref[...], preferred_element_type=jnp.float32)
    o_ref[:, :C0] = _lrelu(h + b_ref[...])
    o_ref[:, C0:C0 + 8] = pos_ref[...]
    o_ref[:, C0 + 8:] = jnp.zeros((x_ref.shape[0], 128 - C0 - 8), jnp.float32)


def _h0cat(x, pos8, W_in, b_in):
    B = 2000
    return pl.pallas_call(
        _h0_body,
        grid=(N // B,),
        in_specs=[
            pl.BlockSpec((B, IN_CHS), lambda i: (i, 0)),
            pl.BlockSpec((B, 8), lambda i: (i, 0)),
            pl.BlockSpec((IN_CHS, C0), lambda i: (0, 0)),
            pl.BlockSpec((1, C0), lambda i: (0, 0)),
        ],
        out_specs=pl.BlockSpec((B, 128), lambda i: (i, 0)),
        out_shape=jax.ShapeDtypeStruct((N, 128), jnp.float32),
    )(x, pos8, W_in, b_in.reshape(1, C0))


_KNN_Q = 200
_NG = 640


def _knn_body(posq_ref, post3_ref, o_ref):
    f32 = jnp.float32
    pq = posq_ref[...]
    sqq = jnp.sum(pq * pq, axis=1, keepdims=True)
    planes = []
    for r in range(K):
        pt = post3_ref[r]
        sqa = jnp.sum(pt * pt, axis=0, keepdims=True)
        mm = jnp.dot(pq, pt, preferred_element_type=f32)
        planes.append((sqq - 2.0 * mm) + sqa)

    def sweep(excl):
        v = jnp.full((_KNN_Q, _NG), jnp.inf, f32)
        rr = jnp.zeros((_KNN_Q, _NG), jnp.int32)
        for r in range(K):
            pr = planes[r]
            for ex in excl:
                pr = jnp.where(ex == r, jnp.inf, pr)
            take = pr < v
            v = jnp.where(take, pr, v)
            rr = jnp.where(take, r, rr)
        return v, rr

    v1, r1 = sweep([])
    v2, r2 = sweep([r1])
    v3, r3 = sweep([r1, r2])
    v4, r4 = sweep([r1, r2, r3])

    iota = lax.broadcasted_iota(jnp.int32, (_KNN_Q, _NG), 1)
    inf = jnp.full((_KNN_Q, _NG), jnp.inf, f32)
    fv, fe = v1, 16 * iota + r1
    q2v, q2e = v2, 16 * iota + r2
    q3v, q3e = v3, 16 * iota + r3
    q4v, q4e = v4, 16 * iota + r4
    for t in range(K):
        am = jnp.argmin(fv, axis=1).astype(jnp.int32)[:, None]
        msk = iota == am
        elem = jnp.min(jnp.where(msk, fe, jnp.int32(2 ** 30)),
                       axis=1, keepdims=True)
        o_ref[:, t:t + 1] = elem
        fv = jnp.where(msk, q2v, fv)
        fe = jnp.where(msk, q2e, fe)
        q2v = jnp.where(msk, q3v, q2v)
        q2e = jnp.where(msk, q3e, q2e)
        q3v = jnp.where(msk, q4v, q3v)
        q3e = jnp.where(msk, q4e, q3e)
        q4v = jnp.where(msk, inf, q4v)

    @pl.when(jnp.any(fv == jnp.inf))
    def _fallback():
        curs = list(planes)
        for t in range(K):
            v = curs[0]
            rr = jnp.zeros((_KNN_Q, _NG), jnp.int32)
            for r in range(1, K):
                take = curs[r] < v
                v = jnp.where(take, curs[r], v)
                rr = jnp.where(take, r, rr)
            am = jnp.argmin(v, axis=1).astype(jnp.int32)[:, None]
            msk = iota == am
            rsel = jnp.min(jnp.where(msk, rr, K), axis=1, keepdims=True)
            o_ref[:, t:t + 1] = 16 * am + rsel
            if t < K - 1:
                for r in range(K):
                    curs[r] = jnp.where(msk & (rsel == r), jnp.inf, curs[r])


def _knn(pos8, posT3):
    return pl.pallas_call(
        _knn_body,
        grid=(N // _KNN_Q,),
        in_specs=[
            pl.BlockSpec((_KNN_Q, 8), lambda i: (i, 0)),
            pl.BlockSpec((K, 8, _NG), lambda i: (0, 0, 0)),
        ],
        out_specs=pl.BlockSpec((_KNN_Q, K), lambda i: (i, 0)),
        out_shape=jax.ShapeDtypeStruct((N, K), jnp.int32),
    )(pos8, posT3)


_GCH = 128
_PER_W = E // 32
_BURST = 4


def _make_gather():
    mesh = plsc.VectorSubcoreMesh(core_axis_name="c", subcore_axis_name="s")

    @functools.partial(
        pl.kernel,
        mesh=mesh,
        out_type=jax.ShapeDtypeStruct((E, 128), jnp.float32),
        scratch_types=[
            pltpu.VMEM((_PER_W,), jnp.int32),
            pltpu.VMEM((_BURST * _GCH, 128), jnp.float32),
            pltpu.SemaphoreType.DMA,
        ],
    )
    def gk(tab_hbm, idx_hbm, out_hbm, idx_v, rows_v, sem):
        wid = lax.axis_index("s") * 2 + lax.axis_index("c")
        base = pl.multiple_of(wid * _PER_W, 8)
        pltpu.sync_copy(idx_hbm.at[pl.ds(base, _PER_W)], idx_v)

        def body(i, carry):
            w = jnp.minimum(i * (_BURST * _GCH), _PER_W - _BURST * _GCH)
            w = pl.multiple_of(w, 8)
            cps = []
            for b in range(_BURST):
                off = pl.multiple_of(w + b * _GCH, 8)
                cps.append(pltpu.async_copy(
                    tab_hbm.at[idx_v.at[pl.ds(off, _GCH)]],
                    rows_v.at[pl.ds(b * _GCH, _GCH)], sem))
            for cp in cps:
                cp.wait()
            pltpu.sync_copy(rows_v, out_hbm.at[pl.ds(base + w, _BURST * _GCH)])
            return carry

        lax.fori_loop(0, (_PER_W + _BURST * _GCH - 1) // (_BURST * _GCH),
                      body, 0)

    return gk


def _layer_body(c, emit_posj, *refs):
    f32 = jnp.float32
    if emit_posj:
        (pos_ref, g_ref, wppi_ref, wppj_ref, wpd_ref, bp_ref, wa11_ref,
         wa12_ref, wa21_ref, wa22_ref, ba1_ref, ba2_ref, wo1_ref, wo2_ref,
         bo_ref, o_ref, posj_out_ref) = refs
        posj_in_ref = None
    else:
        (pos_ref, g_ref, posj_in_ref, wppi_ref, wppj_ref, wpd_ref, bp_ref,
         wa11_ref, wa12_ref, wa21_ref, wa22_ref, ba1_ref, ba2_ref, wo1_ref,
         wo2_ref, bo_ref, o_ref) = refs
        posj_out_ref = None
    pi = pos_ref[...]
    wppi = wppi_ref[...]
    wppj = wppj_ref[...]
    wpd = wpd_ref[...]
    bp = bp_ref[...]
    wa11 = wa11_ref[...]
    wa12 = wa12_ref[...]
    wa21 = wa21_ref[...]
    wa22 = wa22_ref[...]
    ba1 = ba1_ref[...]
    ba2 = ba2_ref[...]

    pes, aas, abs_ = [], [], []
    for j in range(K):
        hj = g_ref[j, :, 0:c]
        if emit_posj:
            pj = g_ref[j, :, c:c + 8]
            posj_out_ref[j] = pj
        else:
            pj = posj_in_ref[j]
        diff = pi - pj
        dist = jnp.sqrt(jnp.sum(diff * diff, axis=1, keepdims=True) + 1e-12)
        pe = _lrelu(jnp.dot(pi, wppi, preferred_element_type=f32)
                    + jnp.dot(pj, wppj, preferred_element_type=f32)
                    + dist * wpd + bp)
        aa = (jnp.dot(hj, wa11, preferred_element_type=f32)
              + jnp.dot(pe, wa21, preferred_element_type=f32) + ba1)
        ab = (jnp.dot(hj, wa12, preferred_element_type=f32)
              + jnp.dot(pe, wa22, preferred_element_type=f32) + ba2)
        pes.append(pe)
        aas.append(aa)
        abs_.append(ab)

    ma = aas[0]
    mb = abs_[0]
    for j in range(1, K):
        ma = jnp.maximum(ma, aas[j])
        mb = jnp.maximum(mb, abs_[j])

    sa = jnp.zeros_like(ma)
    sb = jnp.zeros_like(mb)
    a1 = jnp.zeros_like(ma)
    a2 = jnp.zeros_like(mb)
    for j in range(K):
        ea = jnp.exp(aas[j] - ma)
        eb = jnp.exp(abs_[j] - mb)
        sa = sa + ea
        sb = sb + eb
        a1 = a1 + ea * g_ref[j, :, 0:c]
        a2 = a2 + eb * pes[j]

    agg1 = a1 / (sa + 1e-12)
    agg2 = a2 / (sb + 1e-12)
    out = _lrelu(jnp.dot(agg1, wo1_ref[...], preferred_element_type=f32)
                 + jnp.dot(agg2, wo2_ref[...], preferred_element_type=f32)
                 + bo_ref[...])
    o_ref[...] = out


def _layer(c, emit_posj, pos8, g, posj, Wp, bp, Wa, ba, Wo, bo):
    ch = 2 * c
    B = 400
    wppi = jnp.zeros((8, c), jnp.float32).at[0:3].set(Wp[0:3] + Wp[6:9])
    wppj = jnp.zeros((8, c), jnp.float32).at[0:3].set(Wp[3:6] - Wp[6:9])
    wpd = Wp[9:10]
    body = functools.partial(_layer_body, c, emit_posj)
    wspecs = [
        pl.BlockSpec((8, c), lambda i: (0, 0)),
        pl.BlockSpec((8, c), lambda i: (0, 0)),
        pl.BlockSpec((1, c), lambda i: (0, 0)),
        pl.BlockSpec((1, c), lambda i: (0, 0)),
        pl.BlockSpec((c, c), lambda i: (0, 0)),
        pl.BlockSpec((c, c), lambda i: (0, 0)),
        pl.BlockSpec((c, c), lambda i: (0, 0)),
        pl.BlockSpec((c, c), lambda i: (0, 0)),
        pl.BlockSpec((1, c), lambda i: (0, 0)),
        pl.BlockSpec((1, c), lambda i: (0, 0)),
        pl.BlockSpec((c, ch), lambda i: (0, 0)),
        pl.BlockSpec((c, ch), lambda i: (0, 0)),
        pl.BlockSpec((1, ch), lambda i: (0, 0)),
    ]
    wvals = (wppi, wppj, wpd, bp.reshape(1, c),
             Wa[:c, :c], Wa[:c, c:], Wa[c:, :c], Wa[c:, c:],
             ba[:c].reshape(1, c), ba[c:].reshape(1, c),
             Wo[:c], Wo[c:], bo.reshape(1, ch))
    in_specs = [
        pl.BlockSpec((B, 8), lambda i: (i, 0)),
        pl.BlockSpec((K, B, 128), lambda i: (0, i, 0)),
    ]
    ins = [pos8, g]
    if not emit_posj:
        in_specs.append(pl.BlockSpec((K, B, 8), lambda i: (0, i, 0)))
        ins.append(posj)
    in_specs += wspecs
    ins += wvals
    if emit_posj:
        out_specs = [
            pl.BlockSpec((B, ch), lambda i: (i, 0)),
            pl.BlockSpec((K, B, 8), lambda i: (0, i, 0)),
        ]
        out_shape = [
            jax.ShapeDtypeStruct((N, ch), jnp.float32),
            jax.ShapeDtypeStruct((K, N, 8), jnp.float32),
        ]
    else:
        out_specs = pl.BlockSpec((B, ch), lambda i: (i, 0))
        out_shape = jax.ShapeDtypeStruct((N, ch), jnp.float32)
    return pl.pallas_call(
        body,
        grid=(N // B,),
        in_specs=in_specs,
        out_specs=out_specs,
        out_shape=out_shape,
    )(*ins)


def _final_body(h_ref, x_ref, wmo_ref, wsc_ref, b_ref, o_ref):
    f32 = jnp.float32
    v = (jnp.dot(h_ref[...], wmo_ref[...], preferred_element_type=f32)
         + jnp.dot(x_ref[...], wsc_ref[...], preferred_element_type=f32)
         + b_ref[...])
    o_ref[...] = _lrelu(v)


def _final(h2, x, W_mo, W_sc, bsum):
    B = 2000
    EMB = 512
    return pl.pallas_call(
        _final_body,
        grid=(N // B,),
        in_specs=[
            pl.BlockSpec((B, 256), lambda i: (i, 0)),
            pl.BlockSpec((B, IN_CHS), lambda i: (i, 0)),
            pl.BlockSpec((256, EMB), lambda i: (0, 0)),
            pl.BlockSpec((IN_CHS, EMB), lambda i: (0, 0)),
            pl.BlockSpec((1, EMB), lambda i: (0, 0)),
        ],
        out_specs=pl.BlockSpec((B, EMB), lambda i: (i, 0)),
        out_shape=jax.ShapeDtypeStruct((N, EMB), jnp.float32),
    )(h2, x, W_mo, W_sc, bsum.reshape(1, EMB))


def _gather(tab, idxf):
    return _make_gather()(tab, idxf).reshape(K, N, 128)


def kernel(x, pos, batch, W_in, b_in, W_sc, b_sc, Wp0, bp0, Wa0, ba0, Wo0,
           bo0, Wp1, bp1, Wa1, ba1, Wo1, bo1, W_mo, b_mo):
    pos8 = jnp.pad(pos, ((0, 0), (0, 5)))
    posP = jnp.pad(pos8, ((0, 16 * _NG - N), (0, 0)), constant_values=1e3)
    posT3 = posP.reshape(_NG, K, 8).transpose(1, 2, 0)
    idx = _knn(pos8, posT3)
    idxf = idx.T.reshape(-1)
    h0cat = _h0cat(x, pos8, W_in, b_in)
    g0 = _gather(h0cat, idxf)
    h1, posj = _layer(C0, True, pos8, g0, None, Wp0, bp0, Wa0, ba0, Wo0, bo0)
    g1 = _gather(h1, idxf)
    h2 = _layer(2 * C0, False, pos8, g1, posj, Wp1, bp1, Wa1, ba1, Wo1, bo1)
    return _final(h2, x, W_mo, W_sc, b_mo + b_sc)


# speedup vs baseline: 1.2157x; 1.0221x over previous
import functools

import jax
import jax.numpy as jnp
from jax import lax
from jax.experimental import pallas as pl
from jax.experimental.pallas import tpu as pltpu
from jax.experimental.pallas import tpu_sc as plsc

N = 10000
K = 16
E = N * K
IN_CHS = 128
C0 = 64


def _lrelu(v):
    return jnp.where(v >= 0, v, 0.2 * v)


def _h0_body(x_ref, pos_ref, w_ref, b_ref, o_ref):
    h = jnp.dot(x_ref[...], w_ref[...], preferred_element_type=jnp.float32)
    o_ref[:, :C0] = _lrelu(h + b_ref[...])
    o_ref[:, C0:C0 + 8] = pos_ref[...]
    o_ref[:, C0 + 8:] = jnp.zeros((x_ref.shape[0], 128 - C0 - 8), jnp.float32)


def _h0cat(x, pos8, W_in, b_in):
    B = 2000
    return pl.pallas_call(
        _h0_body,
        grid=(N // B,),
        in_specs=[
            pl.BlockSpec((B, IN_CHS), lambda i: (i, 0)),
            pl.BlockSpec((B, 8), lambda i: (i, 0)),
            pl.BlockSpec((IN_CHS, C0), lambda i: (0, 0)),
            pl.BlockSpec((1, C0), lambda i: (0, 0)),
        ],
        out_specs=pl.BlockSpec((B, 128), lambda i: (i, 0)),
        out_shape=jax.ShapeDtypeStruct((N, 128), jnp.float32),
    )(x, pos8, W_in, b_in.reshape(1, C0))


_KNN_Q = 80
_NG = 640


def _knn_body(posq_ref, post3_ref, o_ref):
    f32 = jnp.float32
    pq = posq_ref[...]
    sqq = jnp.sum(pq * pq, axis=1, keepdims=True)
    planes = []
    for r in range(K):
        pt = post3_ref[r]
        sqa = jnp.sum(pt * pt, axis=0, keepdims=True)
        mm = jnp.dot(pq, pt, preferred_element_type=f32)
        planes.append((sqq - 2.0 * mm) + sqa)

    def sweep(excl):
        v = jnp.full((_KNN_Q, _NG), jnp.inf, f32)
        rr = jnp.zeros((_KNN_Q, _NG), jnp.int32)
        for r in range(K):
            pr = planes[r]
            for ex in excl:
                pr = jnp.where(ex == r, jnp.inf, pr)
            take = pr < v
            v = jnp.where(take, pr, v)
            rr = jnp.where(take, r, rr)
        return v, rr

    v1, r1 = sweep([])
    v2, r2 = sweep([r1])
    v3, r3 = sweep([r1, r2])
    v4, r4 = sweep([r1, r2, r3])

    iota = lax.broadcasted_iota(jnp.int32, (_KNN_Q, _NG), 1)
    inf = jnp.full((_KNN_Q, _NG), jnp.inf, f32)
    fv, fe = v1, 16 * iota + r1
    q2v, q2e = v2, 16 * iota + r2
    q3v, q3e = v3, 16 * iota + r3
    q4v, q4e = v4, 16 * iota + r4
    for t in range(K):
        am = jnp.argmin(fv, axis=1).astype(jnp.int32)[:, None]
        msk = iota == am
        elem = jnp.min(jnp.where(msk, fe, jnp.int32(2 ** 30)),
                       axis=1, keepdims=True)
        o_ref[:, t:t + 1] = elem
        fv = jnp.where(msk, q2v, fv)
        fe = jnp.where(msk, q2e, fe)
        q2v = jnp.where(msk, q3v, q2v)
        q2e = jnp.where(msk, q3e, q2e)
        q3v = jnp.where(msk, q4v, q3v)
        q3e = jnp.where(msk, q4e, q3e)
        q4v = jnp.where(msk, inf, q4v)

    @pl.when(jnp.any(fv == jnp.inf))
    def _fallback():
        curs = list(planes)
        for t in range(K):
            v = curs[0]
            rr = jnp.zeros((_KNN_Q, _NG), jnp.int32)
            for r in range(1, K):
                take = curs[r] < v
                v = jnp.where(take, curs[r], v)
                rr = jnp.where(take, r, rr)
            am = jnp.argmin(v, axis=1).astype(jnp.int32)[:, None]
            msk = iota == am
            rsel = jnp.min(jnp.where(msk, rr, K), axis=1, keepdims=True)
            o_ref[:, t:t + 1] = 16 * am + rsel
            if t < K - 1:
                for r in range(K):
                    curs[r] = jnp.where(msk & (rsel == r), jnp.inf, curs[r])


def _knn(pos8, posT3):
    return pl.pallas_call(
        _knn_body,
        grid=(N // _KNN_Q,),
        in_specs=[
            pl.BlockSpec((_KNN_Q, 8), lambda i: (i, 0)),
            pl.BlockSpec((K, 8, _NG), lambda i: (0, 0, 0)),
        ],
        out_specs=pl.BlockSpec((_KNN_Q, K), lambda i: (i, 0)),
        out_shape=jax.ShapeDtypeStruct((N, K), jnp.int32),
    )(pos8, posT3)


_GCH = 128
_PER_W = E // 32
_BURST = 4


def _make_gather():
    mesh = plsc.VectorSubcoreMesh(core_axis_name="c", subcore_axis_name="s")

    @functools.partial(
        pl.kernel,
        mesh=mesh,
        out_type=jax.ShapeDtypeStruct((E, 128), jnp.float32),
        scratch_types=[
            pltpu.VMEM((_PER_W,), jnp.int32),
            pltpu.VMEM((_BURST * _GCH, 128), jnp.float32),
            pltpu.SemaphoreType.DMA,
        ],
    )
    def gk(tab_hbm, idx_hbm, out_hbm, idx_v, rows_v, sem):
        wid = lax.axis_index("s") * 2 + lax.axis_index("c")
        base = pl.multiple_of(wid * _PER_W, 8)
        pltpu.sync_copy(idx_hbm.at[pl.ds(base, _PER_W)], idx_v)

        def body(i, carry):
            w = jnp.minimum(i * (_BURST * _GCH), _PER_W - _BURST * _GCH)
            w = pl.multiple_of(w, 8)
            cps = []
            for b in range(_BURST):
                off = pl.multiple_of(w + b * _GCH, 8)
                cps.append(pltpu.async_copy(
                    tab_hbm.at[idx_v.at[pl.ds(off, _GCH)]],
                    rows_v.at[pl.ds(b * _GCH, _GCH)], sem))
            for cp in cps:
                cp.wait()
            pltpu.sync_copy(rows_v, out_hbm.at[pl.ds(base + w, _BURST * _GCH)])
            return carry

        lax.fori_loop(0, (_PER_W + _BURST * _GCH - 1) // (_BURST * _GCH),
                      body, 0)

    return gk


def _layer_body(c, emit_posj, *refs):
    f32 = jnp.float32
    if emit_posj:
        (pos_ref, g_ref, wppi_ref, wppj_ref, wpd_ref, bp_ref, wa11_ref,
         wa12_ref, wa21_ref, wa22_ref, ba1_ref, ba2_ref, wo1_ref, wo2_ref,
         bo_ref, o_ref, posj_out_ref) = refs
        posj_in_ref = None
    else:
        (pos_ref, g_ref, posj_in_ref, wppi_ref, wppj_ref, wpd_ref, bp_ref,
         wa11_ref, wa12_ref, wa21_ref, wa22_ref, ba1_ref, ba2_ref, wo1_ref,
         wo2_ref, bo_ref, o_ref) = refs
        posj_out_ref = None
    pi = pos_ref[...]
    wppi = wppi_ref[...]
    wppj = wppj_ref[...]
    wpd = wpd_ref[...]
    bp = bp_ref[...]
    wa11 = wa11_ref[...]
    wa12 = wa12_ref[...]
    wa21 = wa21_ref[...]
    wa22 = wa22_ref[...]
    ba1 = ba1_ref[...]
    ba2 = ba2_ref[...]

    pes, aas, abs_ = [], [], []
    for j in range(K):
        hj = g_ref[j, :, 0:c]
        if emit_posj:
            pj = g_ref[j, :, c:c + 8]
            posj_out_ref[j] = pj
        else:
            pj = posj_in_ref[j]
        diff = pi - pj
        dist = jnp.sqrt(jnp.sum(diff * diff, axis=1, keepdims=True) + 1e-12)
        pe = _lrelu(jnp.dot(pi, wppi, preferred_element_type=f32)
                    + jnp.dot(pj, wppj, preferred_element_type=f32)
                    + dist * wpd + bp)
        aa = (jnp.dot(hj, wa11, preferred_element_type=f32)
              + jnp.dot(pe, wa21, preferred_element_type=f32) + ba1)
        ab = (jnp.dot(hj, wa12, preferred_element_type=f32)
              + jnp.dot(pe, wa22, preferred_element_type=f32) + ba2)
        pes.append(pe)
        aas.append(aa)
        abs_.append(ab)

    ma = aas[0]
    mb = abs_[0]
    for j in range(1, K):
        ma = jnp.maximum(ma, aas[j])
        mb = jnp.maximum(mb, abs_[j])

    sa = jnp.zeros_like(ma)
    sb = jnp.zeros_like(mb)
    a1 = jnp.zeros_like(ma)
    a2 = jnp.zeros_like(mb)
    for j in range(K):
        ea = jnp.exp(aas[j] - ma)
        eb = jnp.exp(abs_[j] - mb)
        sa = sa + ea
        sb = sb + eb
        a1 = a1 + ea * g_ref[j, :, 0:c]
        a2 = a2 + eb * pes[j]

    agg1 = a1 / (sa + 1e-12)
    agg2 = a2 / (sb + 1e-12)
    out = _lrelu(jnp.dot(agg1, wo1_ref[...], preferred_element_type=f32)
                 + jnp.dot(agg2, wo2_ref[...], preferred_element_type=f32)
                 + bo_ref[...])
    o_ref[...] = out


def _layer(c, emit_posj, pos8, g, posj, Wp, bp, Wa, ba, Wo, bo):
    ch = 2 * c
    B = 400
    wppi = jnp.zeros((8, c), jnp.float32).at[0:3].set(Wp[0:3] + Wp[6:9])
    wppj = jnp.zeros((8, c), jnp.float32).at[0:3].set(Wp[3:6] - Wp[6:9])
    wpd = Wp[9:10]
    body = functools.partial(_layer_body, c, emit_posj)
    wspecs = [
        pl.BlockSpec((8, c), lambda i: (0, 0)),
        pl.BlockSpec((8, c), lambda i: (0, 0)),
        pl.BlockSpec((1, c), lambda i: (0, 0)),
        pl.BlockSpec((1, c), lambda i: (0, 0)),
        pl.BlockSpec((c, c), lambda i: (0, 0)),
        pl.BlockSpec((c, c), lambda i: (0, 0)),
        pl.BlockSpec((c, c), lambda i: (0, 0)),
        pl.BlockSpec((c, c), lambda i: (0, 0)),
        pl.BlockSpec((1, c), lambda i: (0, 0)),
        pl.BlockSpec((1, c), lambda i: (0, 0)),
        pl.BlockSpec((c, ch), lambda i: (0, 0)),
        pl.BlockSpec((c, ch), lambda i: (0, 0)),
        pl.BlockSpec((1, ch), lambda i: (0, 0)),
    ]
    wvals = (wppi, wppj, wpd, bp.reshape(1, c),
             Wa[:c, :c], Wa[:c, c:], Wa[c:, :c], Wa[c:, c:],
             ba[:c].reshape(1, c), ba[c:].reshape(1, c),
             Wo[:c], Wo[c:], bo.reshape(1, ch))
    in_specs = [
        pl.BlockSpec((B, 8), lambda i: (i, 0)),
        pl.BlockSpec((K, B, 128), lambda i: (0, i, 0)),
    ]
    ins = [pos8, g]
    if not emit_posj:
        in_specs.append(pl.BlockSpec((K, B, 8), lambda i: (0, i, 0)))
        ins.append(posj)
    in_specs += wspecs
    ins += wvals
    if emit_posj:
        out_specs = [
            pl.BlockSpec((B, ch), lambda i: (i, 0)),
            pl.BlockSpec((K, B, 8), lambda i: (0, i, 0)),
        ]
        out_shape = [
            jax.ShapeDtypeStruct((N, ch), jnp.float32),
            jax.ShapeDtypeStruct((K, N, 8), jnp.float32),
        ]
    else:
        out_specs = pl.BlockSpec((B, ch), lambda i: (i, 0))
        out_shape = jax.ShapeDtypeStruct((N, ch), jnp.float32)
    return pl.pallas_call(
        body,
        grid=(N // B,),
        in_specs=in_specs,
        out_specs=out_specs,
        out_shape=out_shape,
    )(*ins)


def _final_body(h_ref, x_ref, wmo_ref, wsc_ref, b_ref, o_ref):
    f32 = jnp.float32
    v = (jnp.dot(h_ref[...], wmo_ref[...], preferred_element_type=f32)
         + jnp.dot(x_ref[...], wsc_ref[...], preferred_element_type=f32)
         + b_ref[...])
    o_ref[...] = _lrelu(v)


def _final(h2, x, W_mo, W_sc, bsum):
    B = 2000
    EMB = 512
    return pl.pallas_call(
        _final_body,
        grid=(N // B,),
        in_specs=[
            pl.BlockSpec((B, 256), lambda i: (i, 0)),
            pl.BlockSpec((B, IN_CHS), lambda i: (i, 0)),
            pl.BlockSpec((256, EMB), lambda i: (0, 0)),
            pl.BlockSpec((IN_CHS, EMB), lambda i: (0, 0)),
            pl.BlockSpec((1, EMB), lambda i: (0, 0)),
        ],
        out_specs=pl.BlockSpec((B, EMB), lambda i: (i, 0)),
        out_shape=jax.ShapeDtypeStruct((N, EMB), jnp.float32),
    )(h2, x, W_mo, W_sc, bsum.reshape(1, EMB))


def _gather(tab, idxf):
    return _make_gather()(tab, idxf).reshape(K, N, 128)


def kernel(x, pos, batch, W_in, b_in, W_sc, b_sc, Wp0, bp0, Wa0, ba0, Wo0,
           bo0, Wp1, bp1, Wa1, ba1, Wo1, bo1, W_mo, b_mo):
    pos8 = jnp.pad(pos, ((0, 0), (0, 5)))
    posP = jnp.pad(pos8, ((0, 16 * _NG - N), (0, 0)), constant_values=1e3)
    posT3 = posP.reshape(_NG, K, 8).transpose(1, 2, 0)
    idx = _knn(pos8, posT3)
    idxf = idx.T.reshape(-1)
    h0cat = _h0cat(x, pos8, W_in, b_in)
    g0 = _gather(h0cat, idxf)
    h1, posj = _layer(C0, True, pos8, g0, None, Wp0, bp0, Wa0, ba0, Wo0, bo0)
    g1 = _gather(h1, idxf)
    h2 = _layer(2 * C0, False, pos8, g1, posj, Wp1, bp1, Wa1, ba1, Wo1, bo1)
    return _final(h2, x, W_mo, W_sc, b_mo + b_sc)
